# Initial kernel scaffold; baseline (speedup 1.0000x reference)
import jax
import jax.numpy as jnp
from jax.experimental import pallas as pl


def kernel(p, f, voxel_position, voxel_features, We, be, ge, bbe, Ww1, bw1, gw, bw, Ww2, bw2, Wb1, bb1, gb, bb, Wb2, bb2, latent_code, Wl1, bl1, Wl2, bl2, gp, bp):
    raise NotImplementedError("write your pallas kernel here")


# speedup vs baseline: 1.2854x; 1.2854x over previous
import functools

import jax
import jax.numpy as jnp
from jax import lax
from jax.experimental import pallas as pl
from jax.experimental.pallas import tpu as pltpu
from jax.experimental.pallas import tpu_sc as plsc

N = 16384
V = 4096
VC = 64
PC = 64
POSC = 32
LLC = 128
EPS = 1e-5

_BN = 512
_NC, _NS = 2, 16
_NW = _NC * _NS
_BPW = N // _NW
_D = 128


def _argmin_body(p_ref, vpt_ref, idx_ref):
    p = p_ref[...]
    vpt = vpt_ref[...]
    v2 = (vpt[0:1, :] * vpt[0:1, :] + vpt[2:3, :] * vpt[2:3, :]) \
        + vpt[1:2, :] * vpt[1:2, :]
    p2 = (p[:, 0:1] * p[:, 0:1] + p[:, 2:3] * p[:, 2:3]) \
        + p[:, 1:2] * p[:, 1:2]
    dot = jnp.dot(p.astype(jnp.bfloat16), vpt.astype(jnp.bfloat16),
                  preferred_element_type=jnp.float32)
    d2 = (p2 - 2.0 * dot) + v2
    H = V // 8
    ii = lax.broadcasted_iota(jnp.int32, (d2.shape[0], H), 1)
    vs, is_ = [], []
    for c in range(8):
        dc = d2[:, c * H:(c + 1) * H]
        vc = jnp.min(dc, axis=1, keepdims=True)
        ic = jnp.min(jnp.where(dc <= vc, ii, H), axis=1, keepdims=True) + c * H
        vs.append(vc)
        is_.append(ic)
    accv, acci = vs[0], is_[0]
    for c in range(1, 8):
        thr = accv.astype(jnp.bfloat16).astype(jnp.float32)
        t = vs[c] < thr
        accv = jnp.where(t, vs[c], thr)
        acci = jnp.where(t, is_[c], acci)
    idx_ref[...] = acci


def _nn_idx(p, vpt):
    return pl.pallas_call(
        _argmin_body,
        grid=(N // _BN,),
        in_specs=[
            pl.BlockSpec((_BN, 3), lambda i: (i, 0)),
            pl.BlockSpec((3, V), lambda i: (0, 0)),
        ],
        out_specs=pl.BlockSpec((_BN, 1), lambda i: (i, 0)),
        out_shape=jax.ShapeDtypeStruct((N, 1), jnp.int32),
    )(p, vpt)


def _sc_gather(table, idx):
    mesh = plsc.VectorSubcoreMesh(core_axis_name="c", subcore_axis_name="s")

    @functools.partial(
        pl.kernel,
        mesh=mesh,
        out_type=jax.ShapeDtypeStruct((N, _D), jnp.float32),
        scratch_types=[
            pltpu.VMEM((_BPW,), jnp.int32),
            pltpu.VMEM((_BPW, _D), jnp.float32),
            pltpu.SemaphoreType.DMA,
        ],
    )
    def gk(table_hbm, idx_hbm, out_hbm, idx_v, rows_v, sem):
        wid = lax.axis_index("s") * _NC + lax.axis_index("c")
        base = wid * _BPW
        pltpu.sync_copy(idx_hbm.at[pl.ds(base, _BPW)], idx_v)
        pltpu.async_copy(table_hbm.at[idx_v], rows_v, sem).wait()
        pltpu.sync_copy(rows_v, out_hbm.at[pl.ds(base, _BPW)])

    return gk(table, idx)


def _bn_relu(x, g, b):
    m = jnp.mean(x, axis=0, keepdims=True)
    c = x - m
    v = jnp.mean(c * c, axis=0, keepdims=True)
    return jnp.maximum(c / jnp.sqrt(v + EPS) * g + b, 0.0)


def _mlp_body(p_ref, f_ref, vf_ref, vpos_ref,
              We_ref, be_ref, ge_ref, bbe_ref,
              Ww1a_ref, Ww1b_ref, bw1_ref, gw_ref, bw_ref, Ww2_ref, bw2_ref,
              Wb1a_ref, Wb1b_ref, bb1_ref, gb_ref, bb_ref, Wb2_ref, bb2_ref,
              lc_ref, Wl1_ref, bl1_ref, Wl2_ref, bl2_ref,
              gp_ref, bp_ref, out_ref):
    dot = functools.partial(jnp.dot, preferred_element_type=jnp.float32)
    rel = p_ref[...] - vpos_ref[...]
    e = dot(rel, We_ref[...]) + be_ref[...]
    pos = _bn_relu(e, ge_ref[...], bbe_ref[...])
    vf = vf_ref[...]
    uw = dot(vf, Ww1a_ref[...]) + dot(pos, Ww1b_ref[...]) + bw1_ref[...]
    hw = _bn_relu(uw, gw_ref[...], bw_ref[...])
    w_ = dot(hw, Ww2_ref[...]) + bw2_ref[...]
    ub = dot(vf, Wb1a_ref[...]) + dot(pos, Wb1b_ref[...]) + bb1_ref[...]
    hb = _bn_relu(ub, gb_ref[...], bb_ref[...])
    b_ = dot(hb, Wb2_ref[...]) + bb2_ref[...]
    lc = lc_ref[...].reshape(1, LLC)
    hl = jnp.maximum(dot(lc, Wl1_ref[...]) + bl1_ref[...], 0.0)
    s = dot(hl, Wl2_ref[...]) + bl2_ref[...]
    s = s - jnp.max(s, axis=1, keepdims=True)
    es = jnp.exp(s)
    lat = es / jnp.sum(es, axis=1, keepdims=True)
    y = (w_ * lat) * f_ref[...] + (b_ * lat)
    out_ref[...] = _bn_relu(y, gp_ref[...], bp_ref[...])


def _mlp(p, f, vf, vpos, We, be, ge, bbe, Ww1, bw1, gw, bw, Ww2, bw2,
         Wb1, bb1, gb, bb, Wb2, bb2, lc, Wl1, bl1, Wl2, bl2, gp, bp):
    return pl.pallas_call(
        _mlp_body,
        out_shape=jax.ShapeDtypeStruct((N, PC), jnp.float32),
    )(p, f, vf, vpos,
      We, be, ge, bbe,
      Ww1[:VC], Ww1[VC:], bw1, gw, bw, Ww2, bw2,
      Wb1[:VC], Wb1[VC:], bb1, gb, bb, Wb2, bb2,
      lc, Wl1, bl1, Wl2, bl2, gp, bp)


def kernel(p, f, voxel_position, voxel_features,
           We, be, ge, bbe,
           Ww1, bw1, gw, bw, Ww2, bw2,
           Wb1, bb1, gb, bb, Wb2, bb2,
           latent_code, Wl1, bl1, Wl2, bl2,
           gp, bp):
    idx = _nn_idx(p, voxel_position.T).reshape(N)
    table = jnp.concatenate(
        [voxel_features, voxel_position,
         jnp.zeros((V, _D - VC - 3), jnp.float32)], axis=1)
    g = _sc_gather(table, idx)
    vf = g[:, :VC]
    vpos = g[:, VC:VC + 3]
    return _mlp(p, f, vf, vpos, We, be, ge, bbe, Ww1, bw1, gw, bw, Ww2, bw2,
                Wb1, bb1, gb, bb, Wb2, bb2, latent_code, Wl1, bl1, Wl2, bl2,
                gp, bp)
